# baseline (speedup 1.0000x reference)
import functools

import jax
import jax.numpy as jnp
from jax.experimental import pallas as pl
from jax.experimental.pallas import tpu as pltpu


def _fused_mlp_kernel(x_ref, w1_ref, shift_ref, w2_ref, o_ref):
    w1 = w1_ref[...]
    w2 = w2_ref[...]
    shift = shift_ref[...]
    tm = x_ref.shape[2]
    ch = 1024 if tm % 1024 == 0 else tm
    for b in range(x_ref.shape[0]):
        for j in range(tm // ch):
            xc = x_ref[b, :, j * ch:(j + 1) * ch]
            h = jnp.dot(w1, xc, preferred_element_type=jnp.float32)
            h = jnp.maximum(h + shift, 0.0)
            o_ref[b, :, j * ch:(j + 1) * ch] = jnp.dot(
                w2, h, preferred_element_type=jnp.float32)


def _tiling(N, M):
    if M > 4096:
        tm = 4096
        while M % tm:
            tm -= 128
        return 1, max(tm, 128)
    bn = 1
    while bn * 2 <= min(N, 8192 // M * 2) and N % (bn * 2) == 0 and bn < 2:
        bn *= 2
    return bn, M


@functools.partial(jax.jit, static_argnames=("eps",))
def kernel(x_nchw, w1, bn_gamma, bn_beta, bn_mean, bn_var, w2, eps=1e-5):
    N, C, H, W = x_nchw.shape
    NC = w2.shape[0]
    M = H * W

    scale = bn_gamma * jax.lax.rsqrt(bn_var + eps)
    shift = (bn_beta - bn_mean * scale).astype(jnp.float32).reshape(C, 1)
    w1f = (w1.reshape(C, C) * scale[:, None]).astype(jnp.float32)
    w2f = w2.reshape(NC, C).astype(jnp.float32)

    x = x_nchw.reshape(N, C, M)

    BN, TM = _tiling(N, M)
    mt = M // TM
    grid = (pl.cdiv(N, BN) * mt,)

    in_bytes = BN * C * TM * 4
    out_bytes = BN * NC * TM * 4
    vmem_limit = int(min(2 * (in_bytes + out_bytes) + (12 << 20), 100 << 20))

    cost = pl.CostEstimate(
        flops=2 * N * M * (C * C + C * NC),
        transcendentals=0,
        bytes_accessed=N * M * (C + NC) * 4 + (C * C + C + NC * C) * 4,
    )

    out = pl.pallas_call(
        _fused_mlp_kernel,
        out_shape=jax.ShapeDtypeStruct((N, NC, M), x.dtype),
        grid=grid,
        in_specs=[
            pl.BlockSpec((BN, C, TM), lambda i: (i // mt, 0, i % mt)),
            pl.BlockSpec((C, C), lambda i: (0, 0)),
            pl.BlockSpec((C, 1), lambda i: (0, 0)),
            pl.BlockSpec((NC, C), lambda i: (0, 0)),
        ],
        out_specs=pl.BlockSpec((BN, NC, TM), lambda i: (i // mt, 0, i % mt)),
        compiler_params=pltpu.CompilerParams(
            dimension_semantics=("parallel",),
            vmem_limit_bytes=vmem_limit,
        ),
        cost_estimate=cost,
    )(x, w1f, shift, w2f)

    return out.reshape(N, NC, H, W)


# speedup vs baseline: 1.0268x; 1.0268x over previous
import functools

import jax
import jax.numpy as jnp
from jax.experimental import pallas as pl
from jax.experimental.pallas import tpu as pltpu


def _fused_mlp_kernel(x_ref, w1_ref, shift_ref, w2_ref, o_ref):
    w1 = w1_ref[...]
    w2 = w2_ref[...]
    shift = shift_ref[...]
    for b in range(x_ref.shape[0]):
        x = x_ref[b].astype(jnp.bfloat16)
        h = jnp.dot(w1, x, preferred_element_type=jnp.float32)
        h = jnp.maximum(h + shift, 0.0).astype(jnp.bfloat16)
        o_ref[b] = jnp.dot(w2, h, preferred_element_type=jnp.float32)


def _tiling(N, M):
    if M > 4096:
        tm = 4096
        while M % tm:
            tm -= 128
        return 1, max(tm, 128)
    bn = 1
    while bn * 2 <= min(N, 8192 // M * 2) and N % (bn * 2) == 0 and bn < 2:
        bn *= 2
    return bn, M


@functools.partial(jax.jit, static_argnames=("eps",))
def kernel(x_nchw, w1, bn_gamma, bn_beta, bn_mean, bn_var, w2, eps=1e-5):
    N, C, H, W = x_nchw.shape
    NC = w2.shape[0]
    M = H * W

    scale = bn_gamma * jax.lax.rsqrt(bn_var + eps)
    shift = (bn_beta - bn_mean * scale).astype(jnp.float32).reshape(C, 1)
    w1f = (w1.reshape(C, C) * scale[:, None]).astype(jnp.bfloat16)
    w2f = w2.reshape(NC, C).astype(jnp.bfloat16)

    x = x_nchw.reshape(N, C, M)

    BN, TM = _tiling(N, M)
    mt = M // TM
    grid = (pl.cdiv(N, BN) * mt,)

    in_bytes = BN * C * TM * 4
    out_bytes = BN * NC * TM * 4
    vmem_limit = int(min(2 * (in_bytes + out_bytes) + (12 << 20), 100 << 20))

    cost = pl.CostEstimate(
        flops=2 * N * M * (C * C + C * NC),
        transcendentals=0,
        bytes_accessed=N * M * (C + NC) * 4 + (C * C + C + NC * C) * 4,
    )

    out = pl.pallas_call(
        _fused_mlp_kernel,
        out_shape=jax.ShapeDtypeStruct((N, NC, M), x.dtype),
        grid=grid,
        in_specs=[
            pl.BlockSpec((BN, C, TM), lambda i: (i // mt, 0, i % mt)),
            pl.BlockSpec((C, C), lambda i: (0, 0)),
            pl.BlockSpec((C, 1), lambda i: (0, 0)),
            pl.BlockSpec((NC, C), lambda i: (0, 0)),
        ],
        out_specs=pl.BlockSpec((BN, NC, TM), lambda i: (i // mt, 0, i % mt)),
        compiler_params=pltpu.CompilerParams(
            dimension_semantics=("parallel",),
            vmem_limit_bytes=vmem_limit,
        ),
        cost_estimate=cost,
    )(x, w1f, shift, w2f)

    return out.reshape(N, NC, H, W)
